# Initial kernel scaffold; baseline (speedup 1.0000x reference)
import jax
import jax.numpy as jnp
from jax.experimental import pallas as pl


def kernel(t_local, y, edge_index, edge_weight, W_theta, b_lat, W_h, b_unit, W_out):
    raise NotImplementedError("write your pallas kernel here")


# speedup vs baseline: 1.0443x; 1.0443x over previous
import jax
import jax.numpy as jnp
from jax.experimental import pallas as pl

N = 10000
E = 320000
B = 4
LAT = 128
UNITS = 256
NUM_MAT = 5


def _fold_weights(W, d, out):
    V = W.reshape(d, NUM_MAT, out)
    W0 = V[:, 0, :] - V[:, 2, :] - V[:, 4, :]
    return [W0, V[:, 1, :], 2.0 * V[:, 2, :], V[:, 3, :], 2.0 * V[:, 4, :]]


def _spmm(x, src, dst, w):
    return jax.ops.segment_sum(w[:, None] * x[src], dst, num_segments=N)


def _neg_mul_tanh_kernel(theta_ref, pre_ref, o_ref):
    o_ref[...] = -theta_ref[...] * jnp.tanh(pre_ref[...])


def kernel(t_local, y, edge_index, edge_weight, W_theta, b_lat, W_h, b_unit, W_out):
    row = edge_index[0]
    col = edge_index[1]
    ew = edge_weight
    deg1 = jax.ops.segment_sum(ew, row, num_segments=N)
    w1 = ew / jnp.maximum(deg1[row], 1e-12)
    deg2 = jax.ops.segment_sum(ew, col, num_segments=N)
    w2 = ew / jnp.maximum(deg2[col], 1e-12)

    x0 = y.reshape(B, N, LAT).transpose(1, 0, 2).reshape(N, B * LAT)
    r1 = _spmm(x0, row, col, w1)
    r2 = _spmm(r1, row, col, w1)
    r3 = _spmm(x0, col, row, w2)
    r4 = _spmm(r3, col, row, w2)
    feats = [x0, r1, r2, r3, r4]

    Wt = _fold_weights(W_theta, LAT, LAT)
    Wh = _fold_weights(W_h, LAT, UNITS)
    theta_pre = b_lat + sum(
        jnp.einsum("nbi,io->bno", f.reshape(N, B, LAT), Wt[m])
        for m, f in enumerate(feats))
    theta = jax.nn.sigmoid(theta_pre)
    c_pre = b_unit + sum(
        jnp.einsum("nbi,io->bno", f.reshape(N, B, LAT), Wh[m])
        for m, f in enumerate(feats))
    ch = jnp.tanh(c_pre)

    cf = ch.transpose(1, 0, 2).reshape(N, B * UNITS)
    s1 = _spmm(cf, row, col, w1)
    s2 = _spmm(s1, row, col, w1)
    s3 = _spmm(cf, col, row, w2)
    s4 = _spmm(s3, col, row, w2)
    gfeats = [cf, s1, s2, s3, s4]

    Wo = _fold_weights(W_out, UNITS, LAT)
    grad_pre = b_lat + sum(
        jnp.einsum("nbi,io->bno", g.reshape(N, B, UNITS), Wo[m])
        for m, g in enumerate(gfeats))

    grad = pl.pallas_call(
        _neg_mul_tanh_kernel,
        out_shape=jax.ShapeDtypeStruct((B, N, LAT), jnp.float32),
        grid=(B,),
        in_specs=[
            pl.BlockSpec((1, N, LAT), lambda b: (b, 0, 0)),
            pl.BlockSpec((1, N, LAT), lambda b: (b, 0, 0)),
        ],
        out_specs=pl.BlockSpec((1, N, LAT), lambda b: (b, 0, 0)),
    )(theta, grad_pre)
    return grad.reshape(B, N * LAT)


# speedup vs baseline: 2.9504x; 2.8251x over previous
import functools

import jax
import jax.numpy as jnp
from jax import lax
from jax.experimental import pallas as pl
from jax.experimental.pallas import tpu as pltpu
from jax.experimental.pallas import tpu_sc as plsc

N = 10000
E = 320000
B = 4
LAT = 128
UNITS = 256
NUM_MAT = 5

NC = 2
NS = 16
NP = 10240
TPT = NP // NS
EPT = E // NS
K = 80
SB = 25
NSB = EPT // (K * SB)

T = 400
PREC = jax.lax.Precision.HIGHEST

_MESH = dict(core_axis_name="c", subcore_axis_name="s", num_cores=NC,
             num_subcores=NS)


def _sc_weights_body(eidx, ew4, w12, deg_sp, didx_v, ewv, dgv, wv, zv):
    cid = lax.axis_index("c")
    sid = lax.axis_index("s")

    def zwrite(i, _):
        zv[pl.ds(i * 16, 16)] = jnp.zeros((16,), jnp.float32)
        return 0

    lax.fori_loop(0, TPT // 16, zwrite, 0)
    pltpu.sync_copy(zv, deg_sp.at[pl.ds(sid * TPT, TPT)])
    plsc.subcore_barrier()

    def acc_sb(s, _):
        pltpu.sync_copy(eidx.at[cid, sid, s], didx_v)
        pltpu.sync_copy(ew4.at[sid, s], ewv)

        def one(jj, _):
            pltpu.sync_copy(ewv.at[jj], deg_sp.at[didx_v.at[jj]], add=True)
            return 0

        lax.fori_loop(0, SB, one, 0)
        return 0

    lax.fori_loop(0, NSB, acc_sb, 0)
    plsc.subcore_barrier()

    def w_sb(s, _):
        pltpu.sync_copy(eidx.at[cid, sid, s], didx_v)
        pltpu.sync_copy(ew4.at[sid, s], ewv)

        def g1(jj, _):
            pltpu.sync_copy(deg_sp.at[didx_v.at[jj]], dgv.at[jj])
            return 0

        lax.fori_loop(0, SB, g1, 0)

        def cw(jj, _):
            for t in range(K // 16):
                d = dgv[jj, pl.ds(t * 16, 16)]
                e = ewv[jj, pl.ds(t * 16, 16)]
                wv[jj, pl.ds(t * 16, 16)] = e / jnp.maximum(d, 1e-12)
            return 0

        lax.fori_loop(0, SB, cw, 0)
        pltpu.sync_copy(wv, w12.at[cid, sid, s])
        return 0

    lax.fori_loop(0, NSB, w_sb, 0)


def _sc_weights(edge_index, ew):
    eidx = edge_index.reshape(2, NS, NSB, SB, K)
    ew4 = ew.reshape(NS, NSB, SB, K)
    f = pl.kernel(
        _sc_weights_body,
        out_type=jax.ShapeDtypeStruct((2, NS, NSB, SB, K), jnp.float32),
        mesh=plsc.VectorSubcoreMesh(**_MESH),
        scratch_types=[
            pltpu.VMEM_SHARED((NP,), jnp.float32),
            pltpu.VMEM((SB, K), jnp.int32),
            pltpu.VMEM((SB, K), jnp.float32),
            pltpu.VMEM((SB, K), jnp.float32),
            pltpu.VMEM((SB, K), jnp.float32),
            pltpu.VMEM((TPT,), jnp.float32),
        ],
    )
    return f(eidx, ew4)


def _spmm4_body(C, xtab, row4, col4, w14, w24, zrows, out,
                acc, srow_v, sidx2, didx_v, wv, rows0):
    cid = lax.axis_index("c")
    sid = lax.axis_index("s")
    cpc = C // NC

    for step in range(4):
        src4, dst4, w4 = (row4, col4, w14) if step < 2 else (col4, row4, w24)
        gref = xtab if step in (0, 2) else out
        gblock = 0 if step in (0, 2) else (step - 1) * C

        def chunk_body(ci, _, step=step, src4=src4, dst4=dst4, w4=w4,
                       gref=gref, gblock=gblock):
            c = cid * cpc + ci
            gbase = (gblock + c) * NP
            obase = (step * C + c) * NP

            pltpu.sync_copy(zrows, acc.at[pl.ds(sid * TPT, TPT)])
            plsc.subcore_barrier()

            def sb_loop(s, _):
                pltpu.sync_copy(src4.at[sid, s], srow_v)
                pltpu.sync_copy(dst4.at[sid, s], didx_v)
                pltpu.sync_copy(w4.at[sid, s], wv)

                def scale_idx(jj, _):
                    for t in range(K // 16):
                        sidx2[jj, pl.ds(t * 16, 16)] = (
                            srow_v[jj, pl.ds(t * 16, 16)] + gbase)
                    return 0

                lax.fori_loop(0, SB, scale_idx, 0)

                def batch(jj, _):
                    pltpu.sync_copy(gref.at[sidx2.at[jj]], rows0)
                    for t in range(K // 16):
                        w16 = wv[jj, pl.ds(t * 16, 16)]

                        def lloop(l, _, w16=w16, t=t):
                            bidx = jnp.broadcast_to(l, (16,)).astype(jnp.int32)
                            wspl = w16.at[bidx].get(mode="promise_in_bounds")
                            k = t * 16 + l
                            for m in range(8):
                                rows0[k, pl.ds(m * 16, 16)] = (
                                    rows0[k, pl.ds(m * 16, 16)] * wspl)
                            return 0

                        lax.fori_loop(0, 16, lloop, 0)
                    pltpu.sync_copy(rows0, acc.at[didx_v.at[jj]], add=True)
                    return 0

                lax.fori_loop(0, SB, batch, 0)
                return 0

            lax.fori_loop(0, NSB, sb_loop, 0)
            plsc.subcore_barrier()
            pltpu.sync_copy(acc.at[pl.ds(sid * TPT, TPT)],
                            out.at[pl.ds(obase + sid * TPT, TPT)])
            plsc.subcore_barrier()
            return 0

        lax.fori_loop(0, cpc, chunk_body, 0)


def _spmm4(xtab, row4, col4, w14, w24, zrows, C):
    f = pl.kernel(
        functools.partial(_spmm4_body, C),
        out_type=jax.ShapeDtypeStruct((4 * C * NP, 128), jnp.float32),
        mesh=plsc.VectorSubcoreMesh(**_MESH),
        scratch_types=[
            pltpu.VMEM_SHARED((NP, 128), jnp.float32),
            pltpu.VMEM((SB, K), jnp.int32),
            pltpu.VMEM((SB, K), jnp.int32),
            pltpu.VMEM((SB, K), jnp.int32),
            pltpu.VMEM((SB, K), jnp.float32),
            pltpu.VMEM((K, 128), jnp.float32),
        ],
    )
    return f(xtab, row4, col4, w14, w24, zrows)


def _fold_weights(W, d, out):
    V = W.reshape(d, NUM_MAT, out)
    W0 = V[:, 0, :] - V[:, 2, :] - V[:, 4, :]
    return jnp.stack(
        [W0, V[:, 1, :], 2.0 * V[:, 2, :], V[:, 3, :], 2.0 * V[:, 4, :]], 0)


def _tc1_kernel(f0, f1, f2, f3, f4, wt, wh, bl, bu, theta_out, c_out):
    feats = (f0, f1, f2, f3, f4)
    for b in range(B):
        acc_t = jnp.zeros((T, LAT), jnp.float32)
        acc_c = jnp.zeros((T, UNITS), jnp.float32)
        for m in range(NUM_MAT):
            a = feats[m][b, :, :]
            acc_t += jnp.dot(a, wt[m], precision=PREC)
            acc_c += jnp.dot(a, wh[m], precision=PREC)
        theta_out[b, :, :] = jax.nn.sigmoid(acc_t + bl[0, :][None, :])
        cc = acc_c + bu[0, :][None, :]
        c_out[2 * b, :, :] = jnp.tanh(cc[:, :LAT])
        c_out[2 * b + 1, :, :] = jnp.tanh(cc[:, LAT:])


def _tc2_kernel(g0, g1, g2, g3, g4, wo, bl, theta, grad_out):
    gfeats = (g0, g1, g2, g3, g4)
    for b in range(B):
        acc = jnp.zeros((T, LAT), jnp.float32)
        for m in range(NUM_MAT):
            acc += jnp.dot(gfeats[m][2 * b, :, :], wo[m, :LAT, :],
                           precision=PREC)
            acc += jnp.dot(gfeats[m][2 * b + 1, :, :], wo[m, LAT:, :],
                           precision=PREC)
        grad_out[b, :, :] = -theta[b, :, :] * jnp.tanh(acc + bl[0, :][None, :])


def _full(*shape):
    return pl.BlockSpec(shape, lambda nb: (0,) * len(shape))


def kernel(t_local, y, edge_index, edge_weight, W_theta, b_lat, W_h, b_unit, W_out):
    row4 = edge_index[0].reshape(NS, NSB, SB, K)
    col4 = edge_index[1].reshape(NS, NSB, SB, K)
    w12 = _sc_weights(edge_index, edge_weight)
    w14, w24 = w12[0], w12[1]

    zrows = jnp.zeros((TPT, 128), jnp.float32)
    x0p = jnp.pad(y.reshape(B, N, LAT), ((0, 0), (0, NP - N), (0, 0)))
    xtab_y = x0p.reshape(B * NP, 128)

    outy = _spmm4(xtab_y, row4, col4, w14, w24, zrows, 4)
    ry = outy.reshape(4, 4, NP, 128)

    Wt = _fold_weights(W_theta, LAT, LAT)
    Wh = _fold_weights(W_h, LAT, UNITS)

    theta, cf8 = pl.pallas_call(
        _tc1_kernel,
        out_shape=(
            jax.ShapeDtypeStruct((B, N, LAT), jnp.float32),
            jax.ShapeDtypeStruct((2 * B, NP, 128), jnp.float32),
        ),
        grid=(N // T,),
        in_specs=[pl.BlockSpec((B, T, 128), lambda nb: (0, nb, 0))] * 5 + [
            _full(NUM_MAT, LAT, LAT),
            _full(NUM_MAT, LAT, UNITS),
            _full(1, LAT),
            _full(1, UNITS),
        ],
        out_specs=(
            pl.BlockSpec((B, T, LAT), lambda nb: (0, nb, 0)),
            pl.BlockSpec((2 * B, T, 128), lambda nb: (0, nb, 0)),
        ),
    )(x0p, ry[0], ry[1], ry[2], ry[3], Wt, Wh,
      b_lat.reshape(1, LAT), b_unit.reshape(1, UNITS))

    ctab = cf8.reshape(2 * B * NP, 128)
    outc = _spmm4(ctab, row4, col4, w14, w24, zrows, 8)
    sc = outc.reshape(4, 8, NP, 128)

    Wo = _fold_weights(W_out, UNITS, LAT)
    grad = pl.pallas_call(
        _tc2_kernel,
        out_shape=jax.ShapeDtypeStruct((B, N, LAT), jnp.float32),
        grid=(N // T,),
        in_specs=[pl.BlockSpec((2 * B, T, 128), lambda nb: (0, nb, 0))] * 5 + [
            _full(NUM_MAT, UNITS, LAT),
            _full(1, LAT),
            pl.BlockSpec((B, T, LAT), lambda nb: (0, nb, 0)),
        ],
        out_specs=pl.BlockSpec((B, T, LAT), lambda nb: (0, nb, 0)),
    )(cf8, sc[0], sc[1], sc[2], sc[3], Wo, b_lat.reshape(1, LAT), theta)
    return grad.reshape(B, N * LAT)


# speedup vs baseline: 3.1625x; 1.0719x over previous
import functools

import jax
import jax.numpy as jnp
from jax import lax
from jax.experimental import pallas as pl
from jax.experimental.pallas import tpu as pltpu
from jax.experimental.pallas import tpu_sc as plsc

N = 10000
E = 320000
B = 4
LAT = 128
UNITS = 256
NUM_MAT = 5

NC = 2
NS = 16
NP = 10240
TPT = NP // NS
EPT = E // NS
K = 80
NB = EPT // K
BPC = B // NC

SB_A = 25
NSB_A = EPT // (K * SB_A)

T = 400
PREC = jax.lax.Precision.HIGHEST

_MESH = dict(core_axis_name="c", subcore_axis_name="s", num_cores=NC,
             num_subcores=NS)


def _sc_weights_body(eidx, ew4, w12, deg_sp, didx_v, ewv, dgv, wv, zv):
    cid = lax.axis_index("c")
    sid = lax.axis_index("s")

    def zwrite(i, _):
        zv[pl.ds(i * 16, 16)] = jnp.zeros((16,), jnp.float32)
        return 0

    lax.fori_loop(0, TPT // 16, zwrite, 0)
    pltpu.sync_copy(zv, deg_sp.at[pl.ds(sid * TPT, TPT)])
    plsc.subcore_barrier()

    def acc_sb(s, _):
        pltpu.sync_copy(eidx.at[cid, sid, s], didx_v)
        pltpu.sync_copy(ew4.at[sid, s], ewv)

        def one(jj, _):
            pltpu.sync_copy(ewv.at[jj], deg_sp.at[didx_v.at[jj]], add=True)
            return 0

        lax.fori_loop(0, SB_A, one, 0)
        return 0

    lax.fori_loop(0, NSB_A, acc_sb, 0)
    plsc.subcore_barrier()

    def w_sb(s, _):
        pltpu.sync_copy(eidx.at[cid, sid, s], didx_v)
        pltpu.sync_copy(ew4.at[sid, s], ewv)

        def g1(jj, _):
            pltpu.sync_copy(deg_sp.at[didx_v.at[jj]], dgv.at[jj])
            return 0

        lax.fori_loop(0, SB_A, g1, 0)

        def cw(jj, _):
            for t in range(K // 16):
                d = dgv[jj, pl.ds(t * 16, 16)]
                e = ewv[jj, pl.ds(t * 16, 16)]
                wv[jj, pl.ds(t * 16, 16)] = e / jnp.maximum(d, 1e-12)
            return 0

        lax.fori_loop(0, SB_A, cw, 0)
        pltpu.sync_copy(wv, w12.at[cid, sid, s])
        return 0

    lax.fori_loop(0, NSB_A, w_sb, 0)


def _sc_weights(edge_index, ew):
    eidx = edge_index.reshape(2, NS, NSB_A, SB_A, K)
    ew4 = ew.reshape(NS, NSB_A, SB_A, K)
    f = pl.kernel(
        _sc_weights_body,
        out_type=jax.ShapeDtypeStruct((2, NS, NSB_A, SB_A, K), jnp.float32),
        mesh=plsc.VectorSubcoreMesh(**_MESH),
        scratch_types=[
            pltpu.VMEM_SHARED((NP,), jnp.float32),
            pltpu.VMEM((SB_A, K), jnp.int32),
            pltpu.VMEM((SB_A, K), jnp.float32),
            pltpu.VMEM((SB_A, K), jnp.float32),
            pltpu.VMEM((SB_A, K), jnp.float32),
            pltpu.VMEM((TPT,), jnp.float32),
        ],
    )
    return f(eidx, ew4)


def _spmm_stage_body(CI, stage, xtab, rc4, wb4, out,
                     acc, srow_v, didx_v, sidx2, wv, rows):
    cid = lax.axis_index("c")
    sid = lax.axis_index("s")
    CT = CI + 4

    for bi in range(BPC):
        for ch in range(CI):
            dst = (cid * BPC + bi) * CT + ch
            src = (cid * BPC + bi) * CI + ch
            pltpu.sync_copy(xtab.at[pl.ds(src * NP + sid * TPT, TPT)],
                            out.at[pl.ds(dst * NP + sid * TPT, TPT)])
    plsc.subcore_barrier()

    def item(i, _):
        b = cid * BPC + i // 4
        j = lax.rem(i, 4)
        base = b * CT
        d = j // 2
        if stage == 0:
            src_off = jnp.where(j == 0, 0,
                                jnp.where(j == 1, 1,
                                          jnp.where(j == 2, 0, 3)))
            out_off = j + 1
            init_off = jnp.int32(-1)
        else:
            src_off = jnp.where(j == 0, 1,
                                jnp.where(j == 1, 4,
                                          jnp.where(j == 2, 3, 6)))
            out_off = j + 4
            init_off = jnp.where(j == 0, 0, jnp.where(j == 2, 2, -1))
        gc = base + src_off
        oc = base + out_off
        ic = base + jnp.maximum(init_off, 0)
        gvec = jnp.broadcast_to(gc * NP, (16,)).astype(jnp.int32)

        @pl.when(init_off >= 0)
        def _():
            pltpu.sync_copy(out.at[pl.ds(ic * NP + sid * TPT, TPT)],
                            acc.at[pl.ds(sid * TPT, TPT)])

        @pl.when(init_off < 0)
        def _():
            def zr(q, _):
                for m in range(8):
                    rows[q, pl.ds(m * 16, 16)] = jnp.zeros((16,), jnp.float32)
                return 0

            lax.fori_loop(0, K, zr, 0)
            for z in range(TPT // K):
                pltpu.sync_copy(rows, acc.at[pl.ds(sid * TPT + z * K, K)])

        plsc.subcore_barrier()

        def sb_loop(s, _):
            pltpu.sync_copy(rc4.at[d, sid, s], srow_v)
            pltpu.sync_copy(rc4.at[1 - d, sid, s], didx_v)
            pltpu.sync_copy(wb4.at[d, sid, s], wv)

            def mkidx(jj, _):
                for t in range(K // 16):
                    sidx2[jj, pl.ds(t * 16, 16)] = (
                        srow_v[jj, pl.ds(t * 16, 16)] + gvec)
                return 0

            lax.fori_loop(0, SB_A, mkidx, 0)

            def ebatch(jj, _):
                pltpu.sync_copy(out.at[sidx2.at[jj]], rows)
                for t in range(K // 16):
                    w16 = wv[jj, pl.ds(t * 16, 16)]
                    for l in range(16):
                        lvec = jnp.full((16,), l, jnp.int32)
                        wspl = w16.at[lvec].get(mode="promise_in_bounds")
                        k = t * 16 + l
                        for m in range(8):
                            rows[k, pl.ds(m * 16, 16)] = (
                                rows[k, pl.ds(m * 16, 16)] * wspl)
                pltpu.sync_copy(rows, acc.at[didx_v.at[jj]], add=True)
                return 0

            lax.fori_loop(0, SB_A, ebatch, 0)
            return 0

        lax.fori_loop(0, NSB_A, sb_loop, 0)
        plsc.subcore_barrier()
        pltpu.sync_copy(acc.at[pl.ds(sid * TPT, TPT)],
                        out.at[pl.ds(oc * NP + sid * TPT, TPT)])
        plsc.subcore_barrier()
        return 0

    lax.fori_loop(0, 4 * BPC, item, 0)


def _spmm_stage(xtab, rc4, wb4, CI, stage):
    f = pl.kernel(
        functools.partial(_spmm_stage_body, CI, stage),
        out_type=jax.ShapeDtypeStruct((B * (CI + 4) * NP, 128), jnp.float32),
        mesh=plsc.VectorSubcoreMesh(**_MESH),
        scratch_types=[
            pltpu.VMEM_SHARED((NP, 128), jnp.float32),
            pltpu.VMEM((SB_A, K), jnp.int32),
            pltpu.VMEM((SB_A, K), jnp.int32),
            pltpu.VMEM((SB_A, K), jnp.int32),
            pltpu.VMEM((SB_A, K), jnp.float32),
            pltpu.VMEM((K, 128), jnp.float32),
        ],
    )
    return f(xtab, rc4, wb4)


def _fold_weights(W, d, out):
    V = W.reshape(d, NUM_MAT, out)
    W0 = V[:, 0, :] - V[:, 2, :] - V[:, 4, :]
    return jnp.stack(
        [W0, V[:, 1, :], 2.0 * V[:, 2, :], V[:, 3, :], 2.0 * V[:, 4, :]], 0)


def _tc1_kernel(yt, wt, wh, wz, bl, bu, theta_out, z0_out, zc_out):
    for b in range(B):
        acc_t = jnp.zeros((T, LAT), jnp.float32)
        acc_c = jnp.zeros((T, UNITS), jnp.float32)
        for m in range(NUM_MAT):
            a = yt[b, m, :, :]
            acc_t += jnp.dot(a, wt[m], precision=PREC)
            acc_c += jnp.dot(a, wh[m], precision=PREC)
        theta_out[b, :, :] = jax.nn.sigmoid(acc_t + bl[0, :][None, :])
        c = jnp.tanh(acc_c + bu[0, :][None, :])
        z0_out[b, :, :] = jnp.dot(c, wz[0], precision=PREC)
        for m in range(1, NUM_MAT):
            zc_out[b, m - 1, :, :] = jnp.dot(c, wz[m], precision=PREC)


def _tc2_kernel(z0, st, theta, bl, grad_out):
    for b in range(B):
        accv = z0[b, :, :] + st[b, 5, :, :] + st[b, 7, :, :] + bl[0, :][None, :]
        grad_out[b, :, :] = -theta[b, :, :] * jnp.tanh(accv)


def _full(*shape):
    return pl.BlockSpec(shape, lambda nb: (0,) * len(shape))


def kernel(t_local, y, edge_index, edge_weight, W_theta, b_lat, W_h, b_unit, W_out):
    rc4 = edge_index.reshape(2, NS, NSB_A, SB_A, K)
    wb4 = _sc_weights(edge_index, edge_weight)

    x0p = jnp.pad(y.reshape(B, N, LAT), ((0, 0), (0, NP - N), (0, 0)))
    ytab = _spmm_stage(x0p.reshape(B * NP, 128), rc4, wb4, 1, 0)
    yt = ytab.reshape(B, 5, NP, 128)

    Wt = _fold_weights(W_theta, LAT, LAT)
    Wh = _fold_weights(W_h, LAT, UNITS)
    Wo = _fold_weights(W_out, UNITS, LAT)

    theta, z0, zc = pl.pallas_call(
        _tc1_kernel,
        out_shape=(
            jax.ShapeDtypeStruct((B, N, LAT), jnp.float32),
            jax.ShapeDtypeStruct((B, N, LAT), jnp.float32),
            jax.ShapeDtypeStruct((B, 4, NP, 128), jnp.float32),
        ),
        grid=(N // T,),
        in_specs=[
            pl.BlockSpec((B, NUM_MAT, T, 128), lambda nb: (0, 0, nb, 0)),
            _full(NUM_MAT, LAT, LAT),
            _full(NUM_MAT, LAT, UNITS),
            _full(NUM_MAT, UNITS, LAT),
            _full(1, LAT),
            _full(1, UNITS),
        ],
        out_specs=(
            pl.BlockSpec((B, T, LAT), lambda nb: (0, nb, 0)),
            pl.BlockSpec((B, T, LAT), lambda nb: (0, nb, 0)),
            pl.BlockSpec((B, 4, T, 128), lambda nb: (0, 0, nb, 0)),
        ),
    )(yt, Wt, Wh, Wo, b_lat.reshape(1, LAT), b_unit.reshape(1, UNITS))

    stab = _spmm_stage(zc.reshape(B * 4 * NP, 128), rc4, wb4, 4, 1)
    st = stab.reshape(B, 8, NP, 128)

    grad = pl.pallas_call(
        _tc2_kernel,
        out_shape=jax.ShapeDtypeStruct((B, N, LAT), jnp.float32),
        grid=(N // T,),
        in_specs=[
            pl.BlockSpec((B, T, LAT), lambda nb: (0, nb, 0)),
            pl.BlockSpec((B, 8, T, 128), lambda nb: (0, 0, nb, 0)),
            pl.BlockSpec((B, T, LAT), lambda nb: (0, nb, 0)),
            _full(1, LAT),
        ],
        out_specs=pl.BlockSpec((B, T, LAT), lambda nb: (0, nb, 0)),
    )(z0, st, theta, b_lat.reshape(1, LAT))
    return grad.reshape(B, N * LAT)


# speedup vs baseline: 3.7433x; 1.1837x over previous
import functools

import jax
import jax.numpy as jnp
from jax import lax
from jax.experimental import pallas as pl
from jax.experimental.pallas import tpu as pltpu
from jax.experimental.pallas import tpu_sc as plsc

N = 10000
E = 320000
B = 4
LAT = 128
UNITS = 256
NUM_MAT = 5

NC = 2
NS = 16
NP = 10240
TPT = NP // NS
EPT = E // NS
K = 80
NB = EPT // K
BPC = B // NC

SB_A = 25
NSB_A = EPT // (K * SB_A)
SB_B = 10
NSB_B = NB // SB_B
RI2 = SB_B // 2

T = 400
PREC = jax.lax.Precision.HIGHEST

_MESH = dict(core_axis_name="c", subcore_axis_name="s", num_cores=NC,
             num_subcores=NS)


def _sc_weights_body(eidx, ew4, w12, deg_sp, didx_v, ewv, dgv, wv, zv):
    cid = lax.axis_index("c")
    sid = lax.axis_index("s")

    def zwrite(i, _):
        zv[pl.ds(i * 16, 16)] = jnp.zeros((16,), jnp.float32)
        return 0

    lax.fori_loop(0, TPT // 16, zwrite, 0)
    pltpu.sync_copy(zv, deg_sp.at[pl.ds(sid * TPT, TPT)])
    plsc.subcore_barrier()

    def acc_sb(s, _):
        pltpu.sync_copy(eidx.at[cid, sid, s], didx_v)
        pltpu.sync_copy(ew4.at[sid, s], ewv)

        def one(jj, _):
            pltpu.sync_copy(ewv.at[jj], deg_sp.at[didx_v.at[jj]], add=True)
            return 0

        lax.fori_loop(0, SB_A, one, 0)
        return 0

    lax.fori_loop(0, NSB_A, acc_sb, 0)
    plsc.subcore_barrier()

    def w_sb(s, _):
        pltpu.sync_copy(eidx.at[cid, sid, s], didx_v)
        pltpu.sync_copy(ew4.at[sid, s], ewv)

        def g1(jj, _):
            pltpu.sync_copy(deg_sp.at[didx_v.at[jj]], dgv.at[jj])
            return 0

        lax.fori_loop(0, SB_A, g1, 0)

        def cw(jj, _):
            for t in range(K // 16):
                d = dgv[jj, pl.ds(t * 16, 16)]
                e = ewv[jj, pl.ds(t * 16, 16)]
                wv[jj, pl.ds(t * 16, 16)] = e / jnp.maximum(d, 1e-12)
            return 0

        lax.fori_loop(0, SB_A, cw, 0)
        pltpu.sync_copy(wv, w12.at[cid, sid, s])
        return 0

    lax.fori_loop(0, NSB_A, w_sb, 0)


def _sc_weights(edge_index, ew):
    eidx = edge_index.reshape(2, NS, NSB_A, SB_A, K)
    ew4 = ew.reshape(NS, NSB_A, SB_A, K)
    f = pl.kernel(
        _sc_weights_body,
        out_type=jax.ShapeDtypeStruct((2, NS, NSB_A, SB_A, K), jnp.float32),
        mesh=plsc.VectorSubcoreMesh(**_MESH),
        scratch_types=[
            pltpu.VMEM_SHARED((NP,), jnp.float32),
            pltpu.VMEM((SB_A, K), jnp.int32),
            pltpu.VMEM((SB_A, K), jnp.float32),
            pltpu.VMEM((SB_A, K), jnp.float32),
            pltpu.VMEM((SB_A, K), jnp.float32),
            pltpu.VMEM((TPT,), jnp.float32),
        ],
    )
    return f(eidx, ew4)


def _spmm_stage_body(CI, stage, xtab, rc4, wb4, out,
                     acc, srow_v, didx_v, sidx2, wv, rows, gs0, gs1, ss0, ss1):
    cid = lax.axis_index("c")
    sid = lax.axis_index("s")
    CT = CI + 4

    for bi in range(BPC):
        for ch in range(CI):
            dst = (cid * BPC + bi) * CT + ch
            src = (cid * BPC + bi) * CI + ch
            pltpu.sync_copy(xtab.at[pl.ds(src * NP + sid * TPT, TPT)],
                            out.at[pl.ds(dst * NP + sid * TPT, TPT)])
    plsc.subcore_barrier()

    def item(i, _):
        b = cid * BPC + i // 4
        j = lax.rem(i, 4)
        base = b * CT
        d = j // 2
        if stage == 0:
            src_off = jnp.where(j == 0, 0,
                                jnp.where(j == 1, 1,
                                          jnp.where(j == 2, 0, 3)))
            out_off = j + 1
            init_off = jnp.int32(-1)
        else:
            src_off = jnp.where(j == 0, 1,
                                jnp.where(j == 1, 4,
                                          jnp.where(j == 2, 3, 6)))
            out_off = j + 4
            init_off = jnp.where(j == 0, 0, jnp.where(j == 2, 2, -1))
        gc = base + src_off
        oc = base + out_off
        ic = base + jnp.maximum(init_off, 0)
        gvec = jnp.broadcast_to(gc * NP, (16,)).astype(jnp.int32)

        @pl.when(init_off >= 0)
        def _():
            pltpu.sync_copy(out.at[pl.ds(ic * NP + sid * TPT, TPT)],
                            acc.at[pl.ds(sid * TPT, TPT)])

        @pl.when(init_off < 0)
        def _():
            r0z = rows.at[0]

            def zr(q, _):
                for m in range(8):
                    r0z[q, pl.ds(m * 16, 16)] = jnp.zeros((16,), jnp.float32)
                return 0

            lax.fori_loop(0, K, zr, 0)
            for z in range(TPT // K):
                pltpu.sync_copy(r0z, acc.at[pl.ds(sid * TPT + z * K, K)])

        plsc.subcore_barrier()

        def scale(rb, jj):
            def tbody(t, _):
                w16 = wv[jj, pl.ds(t * 16, 16)]
                for l in range(16):
                    lvec = jnp.full((16,), l, jnp.int32)
                    wspl = w16.at[lvec].get(mode="promise_in_bounds")
                    row = t * 16 + l
                    for m in range(8):
                        rb[row, pl.ds(m * 16, 16)] = (
                            rb[row, pl.ds(m * 16, 16)] * wspl)
                return 0

            lax.fori_loop(0, K // 16, tbody, 0)

        r0 = rows.at[0]
        r1 = rows.at[1]

        def sb_loop(s, _):
            pltpu.sync_copy(rc4.at[d, sid, s], srow_v)
            pltpu.sync_copy(rc4.at[1 - d, sid, s], didx_v)
            pltpu.sync_copy(wb4.at[d, sid, s], wv)

            def mkidx(jj, _):
                for t in range(K // 16):
                    sidx2[jj, pl.ds(t * 16, 16)] = (
                        srow_v[jj, pl.ds(t * 16, 16)] + gvec)
                return 0

            lax.fori_loop(0, SB_B, mkidx, 0)

            pltpu.async_copy(out.at[sidx2.at[0]], r0, gs0)

            def q_loop(q, _):
                j0 = 2 * q
                j1 = j0 + 1
                pltpu.make_async_copy(out.at[sidx2.at[j0]], r0, gs0).wait()

                @pl.when(q > 0)
                def _():
                    pltpu.make_async_copy(
                        r1, acc.at[didx_v.at[j1 - 2]], ss1).wait()

                pltpu.async_copy(out.at[sidx2.at[j1]], r1, gs1)
                scale(r0, j0)
                pltpu.async_copy(r0, acc.at[didx_v.at[j0]], ss0, add=True)
                pltpu.make_async_copy(out.at[sidx2.at[j1]], r1, gs1).wait()
                scale(r1, j1)
                pltpu.async_copy(r1, acc.at[didx_v.at[j1]], ss1, add=True)
                pltpu.make_async_copy(r0, acc.at[didx_v.at[j0]], ss0).wait()

                @pl.when(q < RI2 - 1)
                def _():
                    pltpu.async_copy(out.at[sidx2.at[j0 + 2]], r0, gs0)

                return 0

            lax.fori_loop(0, RI2, q_loop, 0)
            pltpu.make_async_copy(r1, acc.at[didx_v.at[SB_B - 1]], ss1).wait()
            return 0

        lax.fori_loop(0, NSB_B, sb_loop, 0)
        plsc.subcore_barrier()
        pltpu.sync_copy(acc.at[pl.ds(sid * TPT, TPT)],
                        out.at[pl.ds(oc * NP + sid * TPT, TPT)])
        plsc.subcore_barrier()
        return 0

    lax.fori_loop(0, 4 * BPC, item, 0)


def _spmm_stage(xtab, rc4, wb4, CI, stage):
    f = pl.kernel(
        functools.partial(_spmm_stage_body, CI, stage),
        out_type=jax.ShapeDtypeStruct((B * (CI + 4) * NP, 128), jnp.float32),
        mesh=plsc.VectorSubcoreMesh(**_MESH),
        scratch_types=[
            pltpu.VMEM_SHARED((NP, 128), jnp.float32),
            pltpu.VMEM((SB_B, K), jnp.int32),
            pltpu.VMEM((SB_B, K), jnp.int32),
            pltpu.VMEM((SB_B, K), jnp.int32),
            pltpu.VMEM((SB_B, K), jnp.float32),
            pltpu.VMEM((2, K, 128), jnp.float32),
            pltpu.SemaphoreType.DMA,
            pltpu.SemaphoreType.DMA,
            pltpu.SemaphoreType.DMA,
            pltpu.SemaphoreType.DMA,
        ],
    )
    return f(xtab, rc4, wb4)


def _fold_weights(W, d, out):
    V = W.reshape(d, NUM_MAT, out)
    W0 = V[:, 0, :] - V[:, 2, :] - V[:, 4, :]
    return jnp.stack(
        [W0, V[:, 1, :], 2.0 * V[:, 2, :], V[:, 3, :], 2.0 * V[:, 4, :]], 0)


def _tc1_kernel(yt, wt, wh, wz, bl, bu, theta_out, z0_out, zc_out):
    for b in range(B):
        acc_t = jnp.zeros((T, LAT), jnp.float32)
        acc_c = jnp.zeros((T, UNITS), jnp.float32)
        for m in range(NUM_MAT):
            a = yt[b, m, :, :]
            acc_t += jnp.dot(a, wt[m], precision=PREC)
            acc_c += jnp.dot(a, wh[m], precision=PREC)
        theta_out[b, :, :] = jax.nn.sigmoid(acc_t + bl[0, :][None, :])
        c = jnp.tanh(acc_c + bu[0, :][None, :])
        z0_out[b, :, :] = jnp.dot(c, wz[0], precision=PREC)
        for m in range(1, NUM_MAT):
            zc_out[b, m - 1, :, :] = jnp.dot(c, wz[m], precision=PREC)


def _tc2_kernel(z0, st, theta, bl, grad_out):
    for b in range(B):
        accv = z0[b, :, :] + st[b, 5, :, :] + st[b, 7, :, :] + bl[0, :][None, :]
        grad_out[b, :, :] = -theta[b, :, :] * jnp.tanh(accv)


def _full(*shape):
    return pl.BlockSpec(shape, lambda nb: (0,) * len(shape))


def kernel(t_local, y, edge_index, edge_weight, W_theta, b_lat, W_h, b_unit, W_out):
    rc4 = edge_index.reshape(2, NS, NSB_B, SB_B, K)
    wb4 = _sc_weights(edge_index, edge_weight).reshape(2, NS, NSB_B, SB_B, K)

    x0p = jnp.pad(y.reshape(B, N, LAT), ((0, 0), (0, NP - N), (0, 0)))
    ytab = _spmm_stage(x0p.reshape(B * NP, 128), rc4, wb4, 1, 0)
    yt = ytab.reshape(B, 5, NP, 128)

    Wt = _fold_weights(W_theta, LAT, LAT)
    Wh = _fold_weights(W_h, LAT, UNITS)
    Wo = _fold_weights(W_out, UNITS, LAT)

    theta, z0, zc = pl.pallas_call(
        _tc1_kernel,
        out_shape=(
            jax.ShapeDtypeStruct((B, N, LAT), jnp.float32),
            jax.ShapeDtypeStruct((B, N, LAT), jnp.float32),
            jax.ShapeDtypeStruct((B, 4, NP, 128), jnp.float32),
        ),
        grid=(N // T,),
        in_specs=[
            pl.BlockSpec((B, NUM_MAT, T, 128), lambda nb: (0, 0, nb, 0)),
            _full(NUM_MAT, LAT, LAT),
            _full(NUM_MAT, LAT, UNITS),
            _full(NUM_MAT, UNITS, LAT),
            _full(1, LAT),
            _full(1, UNITS),
        ],
        out_specs=(
            pl.BlockSpec((B, T, LAT), lambda nb: (0, nb, 0)),
            pl.BlockSpec((B, T, LAT), lambda nb: (0, nb, 0)),
            pl.BlockSpec((B, 4, T, 128), lambda nb: (0, 0, nb, 0)),
        ),
    )(yt, Wt, Wh, Wo, b_lat.reshape(1, LAT), b_unit.reshape(1, UNITS))

    stab = _spmm_stage(zc.reshape(B * 4 * NP, 128), rc4, wb4, 4, 1)
    st = stab.reshape(B, 8, NP, 128)

    grad = pl.pallas_call(
        _tc2_kernel,
        out_shape=jax.ShapeDtypeStruct((B, N, LAT), jnp.float32),
        grid=(N // T,),
        in_specs=[
            pl.BlockSpec((B, T, LAT), lambda nb: (0, nb, 0)),
            pl.BlockSpec((B, 8, T, 128), lambda nb: (0, 0, nb, 0)),
            pl.BlockSpec((B, T, LAT), lambda nb: (0, nb, 0)),
            _full(1, LAT),
        ],
        out_specs=pl.BlockSpec((B, T, LAT), lambda nb: (0, nb, 0)),
    )(z0, st, theta, b_lat.reshape(1, LAT))
    return grad.reshape(B, N * LAT)
